# Initial kernel scaffold; baseline (speedup 1.0000x reference)
import jax
import jax.numpy as jnp
from jax.experimental import pallas as pl


def kernel(x, edge_index, Wl1, bl1, Wr1, g1, b1, Wl2, bl2, Wr2, g2, b2, Wl3, bl3, Wr3):
    raise NotImplementedError("write your pallas kernel here")


# speedup vs baseline: 5.4172x; 5.4172x over previous
import functools
import math

import jax
import jax.numpy as jnp
from jax import lax
from jax.experimental import pallas as pl
from jax.experimental.pallas import tpu as pltpu
from jax.experimental.pallas import tpu_sc as plsc

NC = 2
NS = 16
CH = 128


def _dotT(a, w):
    return lax.dot_general(a, w, (((1,), (1,)), ((), ())),
                           preferred_element_type=jnp.float32)


@functools.partial(jax.jit, static_argnums=(4, 5, 6, 7))
def _sc_segment_sum(y_st, src3, dst3, zrows, n_nodes, w2, nch, zc):
    rpt = n_nodes // NS
    mesh = plsc.VectorSubcoreMesh(core_axis_name="c", subcore_axis_name="s",
                                  num_cores=NC, num_subcores=NS)

    @functools.partial(
        pl.kernel,
        out_type=jax.ShapeDtypeStruct((NC, n_nodes, w2), jnp.float32),
        mesh=mesh,
        scratch_types=[
            pltpu.VMEM((nch, CH), jnp.int32),
            pltpu.VMEM((nch, CH), jnp.int32),
            pltpu.VMEM((CH, w2), jnp.float32),
            pltpu.VMEM((zc, w2), jnp.float32),
            pltpu.VMEM_SHARED((n_nodes + 16, w2), jnp.float32),
            pltpu.SemaphoreType.DMA,
        ],
        compiler_params=pltpu.CompilerParams(use_tc_tiling_on_sc=False),
    )
    def seg(y_hbm, src_hbm, dst_hbm, z_hbm, out_hbm,
            src_v, dst_v, rows_v, zbuf, acc, sem):
        c = lax.axis_index("c")
        s = lax.axis_index("s")
        base = s * rpt
        pltpu.sync_copy(z_hbm, zbuf)
        for k in range(rpt // zc):
            pltpu.sync_copy(zbuf, acc.at[pl.ds(base + k * zc, zc)])
        plsc.subcore_barrier()
        pltpu.sync_copy(src_hbm.at[s], src_v)
        pltpu.sync_copy(dst_hbm.at[s], dst_v)
        tab = y_hbm.at[c]

        def body(j, carry):
            pltpu.async_copy(tab.at[src_v.at[j]], rows_v, sem).wait()
            pltpu.sync_copy(rows_v, acc.at[dst_v.at[j]], add=True)
            return carry

        lax.fori_loop(0, nch, body, 0)
        plsc.subcore_barrier()
        for k in range(rpt // zc):
            off = base + k * zc
            pltpu.sync_copy(acc.at[pl.ds(off, zc)], zbuf)
            pltpu.sync_copy(zbuf, out_hbm.at[c, pl.ds(off, zc)])

    return seg(y_st, src3, dst3, zrows)


def _tc1_body(x_ref, wl_ref, wr_ref, y_ref, z_ref, *, d):
    xb = x_ref[...]
    y = _dotT(xb, wl_ref[...])
    b = xb.shape[0]
    w2 = (d + 32) // 2
    ones = jnp.ones((b, 16), jnp.float32)
    zeros = jnp.zeros((b, 16), jnp.float32)
    lo = y[:, :w2]
    hi = jnp.concatenate([y[:, w2:], ones, zeros], axis=1)
    y_ref[...] = jnp.stack([lo, hi])
    z_ref[...] = _dotT(xb, wr_ref[...])


def _tc2_body(agg_ref, z_ref, wl_ref, wr_ref, bl_ref, s_ref, b_ref,
              y_ref, zo_ref, inv_ref, *, d):
    w2 = (d + 32) // 2
    dcol = d - w2
    alo = agg_ref[0]
    ahi = agg_ref[1]
    deg = ahi[:, dcol:dcol + 1]
    inv = 1.0 / jnp.maximum(deg, 1.0)
    agg = jnp.concatenate([alo, ahi[:, :dcol]], axis=1)
    pre = agg * inv + bl_ref[...] + z_ref[...]
    h = jnp.maximum(pre * s_ref[...] + b_ref[...], 0.0)
    y = _dotT(h, wl_ref[...])
    hh = y.shape[1] // 2
    y_ref[...] = jnp.stack([y[:, :hh], y[:, hh:]])
    zo_ref[...] = _dotT(h, wr_ref[...])
    inv_ref[...] = jnp.broadcast_to(inv, (inv.shape[0], 8))


def _tc3_body(agg_ref, z_ref, inv_ref, wl_ref, wr_ref, bl_ref, s_ref, b_ref,
              y_ref, zo_ref):
    agg = jnp.concatenate([agg_ref[0], agg_ref[1]], axis=1)
    inv = inv_ref[:, :1]
    pre = agg * inv + bl_ref[...] + z_ref[...]
    h = jnp.maximum(pre * s_ref[...] + b_ref[...], 0.0)
    y = _dotT(h, wl_ref[...])
    hh = y.shape[1] // 2
    y_ref[...] = jnp.stack([y[:, :hh], y[:, hh:]])
    zo_ref[...] = _dotT(h, wr_ref[...])


def _tc4_body(agg_ref, z_ref, inv_ref, bl_ref, out_ref, *, c):
    o = jnp.concatenate([agg_ref[0], agg_ref[1]], axis=1)
    inv = inv_ref[:, :1]
    o = o * inv + bl_ref[...] + z_ref[...]
    col = lax.broadcasted_iota(jnp.int32, o.shape, 1)
    om = jnp.where(col < c, o, -jnp.inf)
    m = jnp.max(om, axis=1, keepdims=True)
    e = jnp.exp(om - m)
    ssum = jnp.sum(e, axis=1, keepdims=True)
    out_ref[...] = (o - m - jnp.log(ssum))[:, :c]


def _row_spec(bn, w):
    return pl.BlockSpec((bn, w), lambda i: (i, 0))


def _full_spec(shape):
    nz = (0,) * len(shape)
    return pl.BlockSpec(shape, lambda i: nz)


def _st_spec(bn, w2):
    return pl.BlockSpec((NC, bn, w2), lambda i: (0, i, 0))


def _pad_rows(y_st):
    nc, n, w2 = y_st.shape
    return jnp.concatenate([y_st, jnp.zeros((nc, 8, w2), jnp.float32)], axis=1)


def kernel(x, edge_index, Wl1, bl1, Wr1, g1, b1, Wl2, bl2, Wr2, g2, b2,
           Wl3, bl3, Wr3):
    n, d = x.shape
    h = Wl1.shape[0]
    c = Wl3.shape[0]
    cp = ((c + 31) // 32) * 32
    e = edge_index.shape[1]
    eps = 1e-5

    bn = 1000 if n % 1000 == 0 else n
    grid = (n // bn,)

    rpt = n // NS
    zc = next(z for z in range(min(128, rpt), 0, -1) if rpt % z == 0)

    nch = -(-e // (NS * CH))
    epad = NS * CH * nch
    src = edge_index[0]
    dst = edge_index[1]
    if epad != e:
        fill = jnp.full((epad - e,), n, jnp.int32)
        src = jnp.concatenate([src, fill])
        dst = jnp.concatenate([dst, fill])
    src3 = src.reshape(NS, nch, CH)
    dst3 = dst.reshape(NS, nch, CH)

    s1 = (g1 / math.sqrt(1.0 + eps))[None, :]
    s2 = (g2 / math.sqrt(1.0 + eps))[None, :]
    b1r, b2r = b1[None, :], b2[None, :]
    bl1r, bl2r = bl1[None, :], bl2[None, :]
    Wl3p = jnp.zeros((cp, h), jnp.float32).at[:c].set(Wl3)
    Wr3p = jnp.zeros((cp, h), jnp.float32).at[:c].set(Wr3)
    bl3p = jnp.zeros((1, cp), jnp.float32).at[0, :c].set(bl3)

    w1h = (d + 32) // 2
    z1h = jnp.zeros((zc, w1h), jnp.float32)
    z2h = jnp.zeros((zc, h // 2), jnp.float32)
    z3h = jnp.zeros((zc, cp // 2), jnp.float32)

    y1, zl1 = pl.pallas_call(
        functools.partial(_tc1_body, d=d),
        grid=grid,
        in_specs=[_row_spec(bn, d), _full_spec((h, d)), _full_spec((h, d))],
        out_specs=[_st_spec(bn, w1h), _row_spec(bn, h)],
        out_shape=[jax.ShapeDtypeStruct((NC, n, w1h), jnp.float32),
                   jax.ShapeDtypeStruct((n, h), jnp.float32)],
    )(x, Wl1, Wr1)

    agg1 = _sc_segment_sum(_pad_rows(y1), src3, dst3, z1h, n, w1h, nch, zc)

    y2, zl2, inv8 = pl.pallas_call(
        functools.partial(_tc2_body, d=d),
        grid=grid,
        in_specs=[_st_spec(bn, w1h), _row_spec(bn, h),
                  _full_spec((h, h)), _full_spec((h, h)),
                  _full_spec((1, h)), _full_spec((1, h)), _full_spec((1, h))],
        out_specs=[_st_spec(bn, h // 2), _row_spec(bn, h), _row_spec(bn, 8)],
        out_shape=[jax.ShapeDtypeStruct((NC, n, h // 2), jnp.float32),
                   jax.ShapeDtypeStruct((n, h), jnp.float32),
                   jax.ShapeDtypeStruct((n, 8), jnp.float32)],
    )(agg1, zl1, Wl2, Wr2, bl1r, s1, b1r)

    agg2 = _sc_segment_sum(_pad_rows(y2), src3, dst3, z2h, n, h // 2, nch, zc)

    y3, zl3 = pl.pallas_call(
        _tc3_body,
        grid=grid,
        in_specs=[_st_spec(bn, h // 2), _row_spec(bn, h), _row_spec(bn, 8),
                  _full_spec((cp, h)), _full_spec((cp, h)),
                  _full_spec((1, h)), _full_spec((1, h)), _full_spec((1, h))],
        out_specs=[_st_spec(bn, cp // 2), _row_spec(bn, cp)],
        out_shape=[jax.ShapeDtypeStruct((NC, n, cp // 2), jnp.float32),
                   jax.ShapeDtypeStruct((n, cp), jnp.float32)],
    )(agg2, zl2, inv8, Wl3p, Wr3p, bl2r, s2, b2r)

    agg3 = _sc_segment_sum(_pad_rows(y3), src3, dst3, z3h, n, cp // 2, nch, zc)

    out = pl.pallas_call(
        functools.partial(_tc4_body, c=c),
        grid=grid,
        in_specs=[_st_spec(bn, cp // 2), _row_spec(bn, cp), _row_spec(bn, 8),
                  _full_spec((1, cp))],
        out_specs=_row_spec(bn, c),
        out_shape=jax.ShapeDtypeStruct((n, c), jnp.float32),
    )(agg3, zl3, inv8, bl3p)

    return out
